# Initial kernel scaffold; baseline (speedup 1.0000x reference)
import jax
import jax.numpy as jnp
from jax.experimental import pallas as pl


def kernel(embedding_nodes, encoded_nodes, clusters, cW0, cb0, cW1, cb1, cW2, cb2, nW0, nb0, nW1, nb1, eW0, eb0, eW1, eb1, bn_gamma, bn_beta):
    raise NotImplementedError("write your pallas kernel here")


# speedup vs baseline: 6.2329x; 6.2329x over previous
import jax
import jax.numpy as jnp
from jax.experimental import pallas as pl

N = 50000
C = 512
LAT = 128
HID = 128
EMB = 16
K = 8
B = 2000
NB = N // B
NEG = -3.0e38
POS = 3.0e38
_INTERP = False


def _f32dot(a, b):
    return jnp.dot(a, b, preferred_element_type=jnp.float32)


def _exdot(a, b):
    return jnp.dot(a, b, preferred_element_type=jnp.float32,
                   precision=jax.lax.Precision.HIGHEST)


def _means_from_acc(acc):
    sums = acc[:, :EMB]
    cnt = acc[:, EMB:EMB + 1]
    m = sums / jnp.maximum(cnt, 1.0)
    n = jnp.sqrt(jnp.sum(m * m, axis=1, keepdims=True))
    return m / jnp.maximum(n, 1e-12)


def _k1_body(x_ref, cl_ref, w0_ref, b0_ref, w1_ref, b1_ref, w2_ref, b2_ref,
             emb_ref, acc_ref):
    x = x_ref[...]
    h = jnp.tanh(_f32dot(x, w0_ref[...]) + b0_ref[...])
    h = jnp.tanh(_f32dot(h, w1_ref[...]) + b1_ref[...])
    e = _f32dot(h, w2_ref[...]) + b2_ref[...]
    n = jnp.sqrt(jnp.sum(e * e, axis=1, keepdims=True))
    e = e / jnp.maximum(n, 1e-12)
    emb_ref[...] = e

    cl = cl_ref[0]
    iota_c = jax.lax.broadcasted_iota(jnp.int32, (C, B), 0)
    onehotT = iota_c == cl
    ohf = onehotT.astype(jnp.float32)
    csum = ohf
    d = 1
    while d < B:
        shifted = jnp.concatenate(
            [jnp.zeros((C, d), jnp.float32), csum[:, :B - d]], axis=1)
        csum = csum + shifted
        d *= 2
    rank = jnp.sum(jnp.where(onehotT, csum, 0.0), axis=0, keepdims=True) - 1.0
    rmax = jnp.max(csum[:, B - 1]).astype(jnp.int32)
    padded = jnp.concatenate(
        [e, jnp.ones((B, 1), jnp.float32), jnp.zeros((B, 32 - EMB - 1), jnp.float32)],
        axis=1)

    @pl.when(pl.program_id(0) == 0)
    def _():
        acc_ref[...] = jnp.zeros_like(acc_ref)

    def _round(r, carry):
        sel = jnp.where(onehotT & (rank == r.astype(jnp.float32)), 1.0, 0.0)
        acc_ref[...] += _exdot(sel, padded)
        return carry

    jax.lax.fori_loop(0, rmax, _round, 0, unroll=False)


def _k3_body(acc_ref, emb_ref, topiT_ref, attT_ref, stats_ref):
    means = _means_from_acc(acc_ref[...])
    e = emb_ref[...]
    s = jax.lax.dot_general(means, e, (((1,), (1,)), ((), ())),
                            preferred_element_type=jnp.float32)
    s_hi = jax.lax.dot_general(means, e, (((1,), (1,)), ((), ())),
                               preferred_element_type=jnp.float32,
                               precision=jax.lax.Precision.HIGHEST)
    iota_c = jax.lax.broadcasted_iota(jnp.int32, (C, B), 0)
    tis, tvs = [], []
    for _ in range(K):
        m = jnp.max(s, axis=0, keepdims=True)
        idx = jnp.min(jnp.where(s == m, iota_c, C), axis=0, keepdims=True)
        mk = iota_c == idx
        tis.append(idx)
        tvs.append(jnp.sum(jnp.where(mk, s_hi, 0.0), axis=0, keepdims=True))
        s = jnp.where(mk, NEG, s)
    topiT_ref[0] = jnp.concatenate(tis, axis=0)
    att = jnp.concatenate(tvs, axis=0)
    attT_ref[0] = att

    s1 = jnp.sum(att)
    s2 = jnp.sum(att * att)
    r2 = jax.lax.broadcasted_iota(jnp.int32, (8, 128), 0)
    c2 = jax.lax.broadcasted_iota(jnp.int32, (8, 128), 1)
    blk = (jnp.where((r2 == 0) & (c2 == 0), s1, 0.0)
           + jnp.where((r2 == 1) & (c2 == 0), s2, 0.0))

    @pl.when(pl.program_id(0) == 0)
    def _():
        stats_ref[...] = jnp.zeros_like(stats_ref)

    stats_ref[...] += blk


def _k4_body(x_ref, topiT_ref, attT_ref, stats_ref, w0_ref, b0_ref, w1_ref,
             b1_ref, g_ref, be_ref, logitsT_ref, sn_ref):
    st = stats_ref[...]
    cnt = float(N * K)
    mu = st[0, 0] / cnt
    var = st[1, 0] / cnt - mu * mu
    rstd = jax.lax.rsqrt(var + 1e-5)
    gam = g_ref[0, 0]
    bet = be_ref[0, 0]

    att = attT_ref[0]
    logits = (att - mu) * rstd * gam + bet
    logitsT_ref[0] = logits
    a = jnp.exp(logits)
    den = jnp.sum(a, axis=0, keepdims=True)
    w = a / (1e-12 + den)

    x = x_ref[...]
    nm = jnp.maximum(_f32dot(x, w0_ref[...]) + b0_ref[...], 0.0)
    nm = jnp.maximum(_f32dot(nm, w1_ref[...]) + b1_ref[...], 0.0)

    ti = topiT_ref[0]
    iota_c = jax.lax.broadcasted_iota(jnp.int32, (C, B), 0)
    WT = jnp.zeros((C, B), jnp.float32)
    for k in range(K):
        WT = WT + jnp.where(iota_c == ti[k:k + 1, :], w[k:k + 1, :], 0.0)

    @pl.when(pl.program_id(0) == 0)
    def _():
        sn_ref[...] = jnp.zeros_like(sn_ref)

    sn_ref[...] += _exdot(WT, nm)


def _k5_body(acc_ref, sn_ref, ew0_ref, eb0_ref, ew1_ref, eb1_ref,
             sti_ref, sa_ref, se_ref):
    means = _means_from_acc(acc_ref[...])
    sm = jax.lax.dot_general(means, means, (((1,), (1,)), ((), ())),
                             preferred_element_type=jnp.float32)
    sm_hi = jax.lax.dot_general(means, means, (((1,), (1,)), ((), ())),
                                preferred_element_type=jnp.float32,
                                precision=jax.lax.Precision.HIGHEST)
    iota_l = jax.lax.broadcasted_iota(jnp.int32, (C, C), 1)
    iota_r = jax.lax.broadcasted_iota(jnp.int32, (C, C), 0)
    s = sm
    tis, tvs = [], []
    for _ in range(K):
        m = jnp.max(s, axis=1, keepdims=True)
        idx = jnp.min(jnp.where(s == m, iota_l, C), axis=1, keepdims=True)
        mk = iota_l == idx
        tis.append(idx)
        tvs.append(jnp.sum(jnp.where(mk, sm_hi, 0.0), axis=1, keepdims=True))
        s = jnp.where(mk, NEG, s)
    sti = jnp.concatenate(tis, axis=1)
    vals = jnp.concatenate(tvs, axis=1)
    sti_ref[...] = sti

    masks = [sti[:, k:k + 1] == iota_l for k in range(K)]
    segmax_row = jnp.full((1, C), NEG, jnp.float32)
    segmin_row = jnp.full((1, C), POS, jnp.float32)
    for k in range(K):
        vk = vals[:, k:k + 1]
        segmax_row = jnp.maximum(
            segmax_row, jnp.max(jnp.where(masks[k], vk, NEG), axis=0, keepdims=True))
        segmin_row = jnp.minimum(
            segmin_row, jnp.min(jnp.where(masks[k], vk, POS), axis=0, keepdims=True))
    ident = iota_r == iota_l
    segmax_col = jnp.sum(jnp.where(ident, segmax_row, 0.0), axis=1, keepdims=True)
    segmin_col = jnp.sum(jnp.where(ident, segmin_row, 0.0), axis=1, keepdims=True)
    amax_col = jnp.maximum(segmax_col, jnp.max(vals, axis=1, keepdims=True))
    amin_col = jnp.minimum(segmin_col, jnp.min(vals, axis=1, keepdims=True))
    amax_row = jnp.sum(jnp.where(ident, amax_col, 0.0), axis=0, keepdims=True)
    amin_row = jnp.sum(jnp.where(ident, amin_col, 0.0), axis=0, keepdims=True)

    ga_cols, gi_cols = [], []
    for k in range(K):
        ga_cols.append(jnp.sum(jnp.where(masks[k], amax_row, 0.0), axis=1, keepdims=True))
        gi_cols.append(jnp.sum(jnp.where(masks[k], amin_row, 0.0), axis=1, keepdims=True))
    amax_g1 = jnp.concatenate(ga_cols, axis=1)
    amin_g1 = jnp.concatenate(gi_cols, axis=1)
    sa1 = jnp.tanh(2.0 * (vals - amin_g1) / (1e-12 + (amax_g1 - amin_g1)))
    sa2 = jnp.tanh(2.0 * (vals - amin_col) / (1e-12 + (amax_col - amin_col)))
    sa_ref[0] = sa1
    sa_ref[1] = sa2

    sn = sn_ref[...]
    ew0t = ew0_ref[0:LAT, :]
    ew0b = ew0_ref[LAT:2 * LAT, :]
    eb0 = eb0_ref[...]
    ew1 = ew1_ref[...]
    eb1 = eb1_ref[...]
    A = _f32dot(sn, ew0t)
    Bb = _f32dot(sn, ew0b)
    for k in range(K):
        gk = _exdot(masks[k].astype(jnp.float32), sn)
        h1 = jnp.maximum(A + _f32dot(gk, ew0b) + eb0, 0.0)
        h2 = jnp.maximum(_f32dot(gk, ew0t) + Bb + eb0, 0.0)
        se_ref[k, 0] = jnp.maximum(_f32dot(h1, ew1) + eb1, 0.0)
        se_ref[k, 1] = jnp.maximum(_f32dot(h2, ew1) + eb1, 0.0)


def kernel(embedding_nodes, encoded_nodes, clusters, cW0, cb0, cW1, cb1, cW2,
           cb2, nW0, nb0, nW1, nb1, eW0, eb0, eW1, eb1, bn_gamma, bn_beta):
    f32 = jnp.float32
    clusters3 = clusters.reshape(NB, 1, B)
    cb0r = cb0.reshape(1, HID)
    cb1r = cb1.reshape(1, HID)
    cb2r = cb2.reshape(1, EMB)
    nb0r = nb0.reshape(1, HID)
    nb1r = nb1.reshape(1, LAT)
    eb0r = eb0.reshape(1, HID)
    eb1r = eb1.reshape(1, LAT)
    gamr = jnp.broadcast_to(bn_gamma.reshape(1, 1), (1, 128))
    betr = jnp.broadcast_to(bn_beta.reshape(1, 1), (1, 128))

    emb, acc = pl.pallas_call(
        _k1_body,
        grid=(NB,),
        in_specs=[
            pl.BlockSpec((B, LAT), lambda i: (i, 0)),
            pl.BlockSpec((1, 1, B), lambda i: (i, 0, 0)),
            pl.BlockSpec((LAT, HID), lambda i: (0, 0)),
            pl.BlockSpec((1, HID), lambda i: (0, 0)),
            pl.BlockSpec((HID, HID), lambda i: (0, 0)),
            pl.BlockSpec((1, HID), lambda i: (0, 0)),
            pl.BlockSpec((HID, EMB), lambda i: (0, 0)),
            pl.BlockSpec((1, EMB), lambda i: (0, 0)),
        ],
        out_specs=[
            pl.BlockSpec((B, EMB), lambda i: (i, 0)),
            pl.BlockSpec((C, 32), lambda i: (0, 0)),
        ],
        out_shape=[
            jax.ShapeDtypeStruct((N, EMB), f32),
            jax.ShapeDtypeStruct((C, 32), f32),
        ],
        interpret=_INTERP,
    )(embedding_nodes, clusters3, cW0, cb0r, cW1, cb1r, cW2, cb2r)

    topiT, attT, stats = pl.pallas_call(
        _k3_body,
        grid=(NB,),
        in_specs=[
            pl.BlockSpec((C, 32), lambda i: (0, 0)),
            pl.BlockSpec((B, EMB), lambda i: (i, 0)),
        ],
        out_specs=[
            pl.BlockSpec((1, K, B), lambda i: (i, 0, 0)),
            pl.BlockSpec((1, K, B), lambda i: (i, 0, 0)),
            pl.BlockSpec((8, 128), lambda i: (0, 0)),
        ],
        out_shape=[
            jax.ShapeDtypeStruct((NB, K, B), jnp.int32),
            jax.ShapeDtypeStruct((NB, K, B), f32),
            jax.ShapeDtypeStruct((8, 128), f32),
        ],
        interpret=_INTERP,
    )(acc, emb)

    logitsT, sn = pl.pallas_call(
        _k4_body,
        grid=(NB,),
        in_specs=[
            pl.BlockSpec((B, LAT), lambda i: (i, 0)),
            pl.BlockSpec((1, K, B), lambda i: (i, 0, 0)),
            pl.BlockSpec((1, K, B), lambda i: (i, 0, 0)),
            pl.BlockSpec((8, 128), lambda i: (0, 0)),
            pl.BlockSpec((LAT, HID), lambda i: (0, 0)),
            pl.BlockSpec((1, HID), lambda i: (0, 0)),
            pl.BlockSpec((HID, LAT), lambda i: (0, 0)),
            pl.BlockSpec((1, LAT), lambda i: (0, 0)),
            pl.BlockSpec((1, 128), lambda i: (0, 0)),
            pl.BlockSpec((1, 128), lambda i: (0, 0)),
        ],
        out_specs=[
            pl.BlockSpec((1, K, B), lambda i: (i, 0, 0)),
            pl.BlockSpec((C, LAT), lambda i: (0, 0)),
        ],
        out_shape=[
            jax.ShapeDtypeStruct((NB, K, B), f32),
            jax.ShapeDtypeStruct((C, LAT), f32),
        ],
        interpret=_INTERP,
    )(encoded_nodes, topiT, attT, stats, nW0, nb0r, nW1, nb1r, gamr, betr)

    sti, sa, seo = pl.pallas_call(
        _k5_body,
        out_shape=[
            jax.ShapeDtypeStruct((C, K), jnp.int32),
            jax.ShapeDtypeStruct((2, C, K), f32),
            jax.ShapeDtypeStruct((K, 2, C, LAT), f32),
        ],
        interpret=_INTERP,
    )(acc, sn, eW0, eb0r, eW1, eb1r)

    bg0 = jnp.repeat(jnp.arange(N, dtype=jnp.int32), K)
    bg1 = topiT.transpose(0, 2, 1).reshape(-1)
    bipartite_graph = jnp.stack([bg0, bg1])
    logits = logitsT.transpose(0, 2, 1).reshape(-1)

    s_src = jnp.repeat(jnp.arange(C, dtype=jnp.int32), K)
    s_dst = sti.reshape(-1)
    super_graph = jnp.stack([jnp.concatenate([s_src, s_dst]),
                             jnp.concatenate([s_dst, s_src])])
    super_att = sa.reshape(-1)[:, None]
    se = seo.transpose(1, 2, 0, 3).reshape(2 * C * K, LAT)

    return (emb, sn, se, bipartite_graph, logits, super_graph, super_att)
